# Initial kernel scaffold; baseline (speedup 1.0000x reference)
import jax
import jax.numpy as jnp
from jax.experimental import pallas as pl


def kernel(node_features, edge_features, radial_embedding, senders, receivers, node_species, W_up, W_r1, W_r2, W_down0, W_down1, W_down2, W_skip):
    raise NotImplementedError("write your pallas kernel here")


# speedup vs baseline: 38.2382x; 38.2382x over previous
import functools

import jax
import jax.numpy as jnp
from jax import lax
from jax.experimental import pallas as pl
from jax.experimental.pallas import tpu as pltpu
from jax.experimental.pallas import tpu_sc as plsc

_INV_SQRT_AVG_NEIGH = 1.0 / (16.0 ** 0.5)


def _node_matmuls(nf, w_up, w_skip0):
    n, d = nf.shape
    ks = w_skip0.shape[1]
    cn = 1000
    assert n % cn == 0

    def body(nf_ref, wu_ref, wsk_ref, h_ref, skip_ref):
        x = nf_ref[...]
        h_ref[...] = jnp.dot(x, wu_ref[...], preferred_element_type=jnp.float32)
        skip_ref[...] = jnp.dot(x, wsk_ref[...], preferred_element_type=jnp.float32)

    return pl.pallas_call(
        body,
        grid=(n // cn,),
        in_specs=[
            pl.BlockSpec((cn, d), lambda i: (i, 0)),
            pl.BlockSpec((d, d), lambda i: (0, 0)),
            pl.BlockSpec((d, ks), lambda i: (0, 0)),
        ],
        out_specs=[
            pl.BlockSpec((cn, d), lambda i: (i, 0)),
            pl.BlockSpec((cn, ks), lambda i: (i, 0)),
        ],
        out_shape=[
            jax.ShapeDtypeStruct((n, d), jnp.float32),
            jax.ShapeDtypeStruct((n, ks), jnp.float32),
        ],
    )(nf, w_up, w_skip0)


def _sc_gather(h, senders):
    n, d = h.shape
    e = senders.shape[0]
    nw = 32
    epw = e // nw
    sup = 1000
    ch = 40
    assert e % nw == 0 and epw % sup == 0 and sup % ch == 0

    mesh = plsc.VectorSubcoreMesh(core_axis_name="c", subcore_axis_name="s")

    @functools.partial(
        pl.kernel,
        out_type=jax.ShapeDtypeStruct((e, d), jnp.float32),
        mesh=mesh,
        scratch_types=[
            pltpu.VMEM((sup,), jnp.int32),
            pltpu.VMEM((ch, d), jnp.float32),
        ],
    )
    def k(h_hbm, s_hbm, out_hbm, idx_v, rows_v):
        c = lax.axis_index("c")
        s = lax.axis_index("s")
        wid = s * 2 + c
        base = wid * epw

        def sup_body(g, carry):
            sb = base + g * sup
            pltpu.sync_copy(s_hbm.at[pl.ds(sb, sup)], idx_v)

            def ch_body(j, carry2):
                pltpu.sync_copy(h_hbm.at[idx_v.at[pl.ds(j * ch, ch)]], rows_v)
                pltpu.sync_copy(rows_v, out_hbm.at[pl.ds(sb + j * ch, ch)])
                return carry2

            return lax.fori_loop(0, sup // ch, ch_body, carry)

        lax.fori_loop(0, epw // sup, sup_body, 0)

    return k(h, senders)


def _edge_payload(rad, ef, hs, w_r1, w_r2, w_d0, w_d1, w_d2):
    e, r = rad.shape
    d = hs.shape[1]
    ce = 2000
    assert e % ce == 0

    def body(rad_ref, ef_ref, hs_ref, wr1_ref, wr2_ref, wd0_ref, wd1_ref,
             wd2_ref, pay_ref):
        radb = rad_ref[...]
        efb = ef_ref[...]
        hsb = hs_ref[...]
        hid = jnp.dot(radb, wr1_ref[...], preferred_element_type=jnp.float32)
        hid = hid * jax.nn.sigmoid(hid)
        w = jnp.dot(hid, wr2_ref[...], preferred_element_type=jnp.float32)
        p0 = w[:, 0:d] * hsb
        p1 = w[:, d:2 * d] * hsb
        p2 = w[:, 2 * d:3 * d] * hsb
        u0 = jnp.dot(p0, wd0_ref[...], preferred_element_type=jnp.float32)
        u0 = u0 * efb[:, 0:1]
        u1 = jnp.dot(p1, wd1_ref[...], preferred_element_type=jnp.float32)
        u2 = jnp.dot(p2, wd2_ref[...], preferred_element_type=jnp.float32)
        v = [u1 * efb[:, 1 + i:2 + i] for i in range(3)]
        t = [u2 * efb[:, 4 + i:5 + i] for i in range(5)]
        pay_ref[0] = jnp.concatenate([u0, v[0]], axis=1)
        pay_ref[1] = jnp.concatenate([v[1], v[2], t[0], t[1]], axis=1)
        pay_ref[2] = jnp.concatenate(
            [t[2], t[3], t[4], jnp.zeros((u0.shape[0], 32), jnp.float32)], axis=1)

    return pl.pallas_call(
        body,
        grid=(e // ce,),
        in_specs=[
            pl.BlockSpec((ce, r), lambda i: (i, 0)),
            pl.BlockSpec((ce, 9), lambda i: (i, 0)),
            pl.BlockSpec((ce, d), lambda i: (i, 0)),
            pl.BlockSpec((r, 8), lambda i: (0, 0)),
            pl.BlockSpec((8, 3 * d), lambda i: (0, 0)),
            pl.BlockSpec((d, 96), lambda i: (0, 0)),
            pl.BlockSpec((d, 32), lambda i: (0, 0)),
            pl.BlockSpec((d, 32), lambda i: (0, 0)),
        ],
        out_specs=pl.BlockSpec((3, ce, 128), lambda i: (0, i, 0)),
        out_shape=jax.ShapeDtypeStruct((3, e, 128), jnp.float32),
    )(rad, ef, hs, w_r1, w_r2, w_d0, w_d1, w_d2)


def _sc_scatter(pay, recv, zeros_init, n):
    ng, e, w = pay.shape
    ch = 128
    ns = 16
    epc = e // 2
    n_ch = epc // ch
    npt = n // ns
    assert e % (2 * ch) == 0 and n % ns == 0 and npt % 8 == 0

    mesh = plsc.VectorSubcoreMesh(core_axis_name="c", subcore_axis_name="s")

    @functools.partial(
        pl.kernel,
        out_type=jax.ShapeDtypeStruct((2, ng, n, w), jnp.float32),
        mesh=mesh,
        scratch_types=[
            pltpu.VMEM_SHARED((n, w), jnp.float32),
            pltpu.VMEM((ch,), jnp.int32),
            pltpu.VMEM((ch, w), jnp.float32),
        ],
    )
    def k(pay_hbm, recv_hbm, zero_hbm, out_hbm, acc, idxc, payb):
        c = lax.axis_index("c")
        s = lax.axis_index("s")
        ebase = c * epc
        trips = (n_ch - s + ns - 1) // ns

        for g in range(ng):
            pltpu.sync_copy(zero_hbm, acc.at[pl.ds(s * npt, npt)])
            plsc.subcore_barrier()

            def ch_body(t, carry, g=g):
                eb = ebase + (s + t * ns) * ch
                pltpu.sync_copy(recv_hbm.at[pl.ds(eb, ch)], idxc)
                pltpu.sync_copy(pay_hbm.at[g, pl.ds(eb, ch)], payb)
                pltpu.sync_copy(payb, acc.at[idxc], add=True)
                return carry

            lax.fori_loop(0, trips, ch_body, 0)
            plsc.subcore_barrier()
            pltpu.sync_copy(acc.at[pl.ds(s * npt, npt)],
                            out_hbm.at[c, g, pl.ds(s * npt, npt)])
            plsc.subcore_barrier()

    return k(pay, recv, zeros_init)


def _gate(acc, skip, n):
    cn = 1000
    assert n % cn == 0

    def body(acc_ref, skip_ref, s_ref, v_ref, t_ref):
        a = acc_ref[0] + acc_ref[1]
        a0, a1, a2 = a[0], a[1], a[2]
        seg_s = a0[:, 0:96]
        s = 0.5 * (seg_s * _INV_SQRT_AVG_NEIGH + skip_ref[...])
        scal = s[:, 0:32]
        g1 = s[:, 32:64]
        g2 = s[:, 64:96]
        s_ref[...] = scal * jax.nn.sigmoid(scal)
        v_sh = jnp.concatenate([a0[:, 96:128], a1[:, 0:64]], axis=1)
        t_sh = jnp.concatenate([a1[:, 64:128], a2[:, 0:96]], axis=1)
        gate1 = g1 * jax.nn.sigmoid(g1)
        gate2 = g2 * jax.nn.sigmoid(g2)
        half_inv = 0.5 * _INV_SQRT_AVG_NEIGH
        v_ref[...] = (v_sh * half_inv) * jnp.concatenate([gate1] * 3, axis=1)
        t_ref[...] = (t_sh * half_inv) * jnp.concatenate([gate2] * 5, axis=1)

    return pl.pallas_call(
        body,
        grid=(n // cn,),
        in_specs=[
            pl.BlockSpec((2, 3, cn, 128), lambda i: (0, 0, i, 0)),
            pl.BlockSpec((cn, 96), lambda i: (i, 0)),
        ],
        out_specs=[
            pl.BlockSpec((cn, 32), lambda i: (i, 0)),
            pl.BlockSpec((cn, 96), lambda i: (i, 0)),
            pl.BlockSpec((cn, 160), lambda i: (i, 0)),
        ],
        out_shape=[
            jax.ShapeDtypeStruct((n, 32), jnp.float32),
            jax.ShapeDtypeStruct((n, 96), jnp.float32),
            jax.ShapeDtypeStruct((n, 160), jnp.float32),
        ],
    )(acc, skip)


def kernel(node_features, edge_features, radial_embedding, senders, receivers,
           node_species, W_up, W_r1, W_r2, W_down0, W_down1, W_down2, W_skip):
    n, d = node_features.shape
    e = senders.shape[0]
    del node_species

    h, skip = _node_matmuls(node_features, W_up, W_skip[0])
    hs = _sc_gather(h, senders.astype(jnp.int32))
    pay = _edge_payload(radial_embedding, edge_features, hs,
                        W_r1, W_r2, W_down0, W_down1, W_down2)
    n_pad = 10240
    zeros_init = jnp.zeros((n_pad // 16, 128), jnp.float32)
    acc = _sc_scatter(pay, receivers.astype(jnp.int32), zeros_init, n_pad)
    out_s, out_v_sh, out_t_sh = _gate(acc, skip, n)

    out_v = out_v_sh.reshape(n, 3, 32).transpose(0, 2, 1).reshape(n, 96)
    out_t = out_t_sh.reshape(n, 5, 32).transpose(0, 2, 1).reshape(n, 160)
    return jnp.concatenate([out_s, out_v, out_t], axis=1)


# speedup vs baseline: 43.6881x; 1.1425x over previous
import functools

import jax
import jax.numpy as jnp
from jax import lax
from jax.experimental import pallas as pl
from jax.experimental.pallas import tpu as pltpu
from jax.experimental.pallas import tpu_sc as plsc

_INV_SQRT_AVG_NEIGH = 1.0 / (16.0 ** 0.5)


def _node_matmuls(nf, w_up, w_skip0):
    n, d = nf.shape
    ks = w_skip0.shape[1]
    cn = 1000
    assert n % cn == 0

    def body(nf_ref, wu_ref, wsk_ref, h_ref, skip_ref):
        x = nf_ref[...]
        h_ref[...] = jnp.dot(x, wu_ref[...], preferred_element_type=jnp.float32)
        skip_ref[...] = jnp.dot(x, wsk_ref[...], preferred_element_type=jnp.float32)

    return pl.pallas_call(
        body,
        grid=(n // cn,),
        in_specs=[
            pl.BlockSpec((cn, d), lambda i: (i, 0)),
            pl.BlockSpec((d, d), lambda i: (0, 0)),
            pl.BlockSpec((d, ks), lambda i: (0, 0)),
        ],
        out_specs=[
            pl.BlockSpec((cn, d), lambda i: (i, 0)),
            pl.BlockSpec((cn, ks), lambda i: (i, 0)),
        ],
        out_shape=[
            jax.ShapeDtypeStruct((n, d), jnp.float32),
            jax.ShapeDtypeStruct((n, ks), jnp.float32),
        ],
    )(nf, w_up, w_skip0)


def _sc_gather(h, senders):
    n, d = h.shape
    e = senders.shape[0]
    nw = 32
    epw = e // nw
    sup = 1000
    ch = 40
    assert e % nw == 0 and epw % sup == 0 and sup % ch == 0

    mesh = plsc.VectorSubcoreMesh(core_axis_name="c", subcore_axis_name="s")

    @functools.partial(
        pl.kernel,
        out_type=jax.ShapeDtypeStruct((e, d), jnp.float32),
        mesh=mesh,
        scratch_types=[
            pltpu.VMEM((sup,), jnp.int32),
            pltpu.VMEM((ch, d), jnp.float32),
        ],
    )
    def k(h_hbm, s_hbm, out_hbm, idx_v, rows_v):
        c = lax.axis_index("c")
        s = lax.axis_index("s")
        wid = s * 2 + c
        base = wid * epw

        def sup_body(g, carry):
            sb = base + g * sup
            pltpu.sync_copy(s_hbm.at[pl.ds(sb, sup)], idx_v)

            def ch_body(j, carry2):
                pltpu.sync_copy(h_hbm.at[idx_v.at[pl.ds(j * ch, ch)]], rows_v)
                pltpu.sync_copy(rows_v, out_hbm.at[pl.ds(sb + j * ch, ch)])
                return carry2

            return lax.fori_loop(0, sup // ch, ch_body, carry)

        lax.fori_loop(0, epw // sup, sup_body, 0)

    return k(h, senders)


def _edge_payload(rad, ef, hs, w_r1, w_r2, w_d0, w_d1, w_d2):
    e, r = rad.shape
    d = hs.shape[1]
    ce = 2000
    assert e % ce == 0

    bf = jnp.bfloat16
    zero_d32 = jnp.zeros((d, 32), jnp.float32)
    zero_d96 = jnp.zeros((d, 96), jnp.float32)
    w_a = jnp.concatenate([w_d1, w_d1, w_d1, zero_d32], axis=1).astype(bf)
    w_b = jnp.concatenate([zero_d96, w_d2], axis=1).astype(bf)
    w_c = jnp.concatenate([w_d2, w_d2, w_d2, w_d2], axis=1).astype(bf)
    w_dd = jnp.concatenate([w_d0, zero_d32], axis=1).astype(bf)
    m = jnp.zeros((9, 384), jnp.float32)
    for i in range(3):
        m = m.at[1 + i, 32 * i:32 * (i + 1)].set(1.0)
    m = m.at[8, 96:128].set(1.0)
    for i in range(4):
        m = m.at[4 + i, 128 + 32 * i:160 + 32 * i].set(1.0)
    m = m.at[0, 256:352].set(1.0)

    def body(rad_ref, ef_ref, hs_ref, wr1_ref, wr2_ref, wa_ref, wb_ref,
             wc_ref, wd_ref, m_ref, pay_ref):
        radb = rad_ref[...]
        efb = ef_ref[...]
        hsb = hs_ref[...]
        hid = jnp.dot(radb, wr1_ref[...], preferred_element_type=jnp.float32)
        hid = hid * jax.nn.sigmoid(hid)
        w = jnp.dot(hid.astype(bf), wr2_ref[...].astype(bf),
                    preferred_element_type=jnp.float32)
        p0 = (w[:, 0:d] * hsb).astype(bf)
        p1 = (w[:, d:2 * d] * hsb).astype(bf)
        p2 = (w[:, 2 * d:3 * d] * hsb).astype(bf)
        shb = jnp.dot(efb, m_ref[...], preferred_element_type=jnp.float32)
        g0 = jnp.dot(p1, wa_ref[...], preferred_element_type=jnp.float32)
        g0 = g0 + jnp.dot(p2, wb_ref[...], preferred_element_type=jnp.float32)
        g1 = jnp.dot(p2, wc_ref[...], preferred_element_type=jnp.float32)
        g2 = jnp.dot(p0, wd_ref[...], preferred_element_type=jnp.float32)
        pay_ref[0] = g0 * shb[:, 0:128]
        pay_ref[1] = g1 * shb[:, 128:256]
        pay_ref[2] = g2 * shb[:, 256:384]

    return pl.pallas_call(
        body,
        grid=(e // ce,),
        in_specs=[
            pl.BlockSpec((ce, r), lambda i: (i, 0)),
            pl.BlockSpec((ce, 9), lambda i: (i, 0)),
            pl.BlockSpec((ce, d), lambda i: (i, 0)),
            pl.BlockSpec((r, 8), lambda i: (0, 0)),
            pl.BlockSpec((8, 3 * d), lambda i: (0, 0)),
            pl.BlockSpec((d, 128), lambda i: (0, 0)),
            pl.BlockSpec((d, 128), lambda i: (0, 0)),
            pl.BlockSpec((d, 128), lambda i: (0, 0)),
            pl.BlockSpec((d, 128), lambda i: (0, 0)),
            pl.BlockSpec((9, 384), lambda i: (0, 0)),
        ],
        out_specs=pl.BlockSpec((3, ce, 128), lambda i: (0, i, 0)),
        out_shape=jax.ShapeDtypeStruct((3, e, 128), jnp.float32),
    )(rad, ef, hs, w_r1, w_r2, w_a, w_b, w_c, w_dd, m)


def _sc_scatter(pay, recv, zeros_init, n):
    ng, e, w = pay.shape
    ch = 128
    ns = 16
    epc = e // 2
    n_ch = epc // ch
    npt = n // ns
    assert e % (2 * ch) == 0 and n % ns == 0 and npt % 8 == 0

    mesh = plsc.VectorSubcoreMesh(core_axis_name="c", subcore_axis_name="s")

    @functools.partial(
        pl.kernel,
        out_type=jax.ShapeDtypeStruct((2, ng, n, w), jnp.float32),
        mesh=mesh,
        scratch_types=[
            pltpu.VMEM_SHARED((n, w), jnp.float32),
            pltpu.VMEM((ch,), jnp.int32),
            pltpu.VMEM((ch, w), jnp.float32),
        ],
    )
    def k(pay_hbm, recv_hbm, zero_hbm, out_hbm, acc, idxc, payb):
        c = lax.axis_index("c")
        s = lax.axis_index("s")
        ebase = c * epc
        trips = (n_ch - s + ns - 1) // ns

        for g in range(ng):
            pltpu.sync_copy(zero_hbm, acc.at[pl.ds(s * npt, npt)])
            plsc.subcore_barrier()

            def ch_body(t, carry, g=g):
                eb = ebase + (s + t * ns) * ch
                pltpu.sync_copy(recv_hbm.at[pl.ds(eb, ch)], idxc)
                pltpu.sync_copy(pay_hbm.at[g, pl.ds(eb, ch)], payb)
                pltpu.sync_copy(payb, acc.at[idxc], add=True)
                return carry

            lax.fori_loop(0, trips, ch_body, 0)
            plsc.subcore_barrier()
            pltpu.sync_copy(acc.at[pl.ds(s * npt, npt)],
                            out_hbm.at[c, g, pl.ds(s * npt, npt)])
            plsc.subcore_barrier()

    return k(pay, recv, zeros_init)


def _gate(acc, skip, n):
    cn = 1000
    assert n % cn == 0

    def body(acc_ref, skip_ref, s_ref, v_ref, t_ref):
        a = acc_ref[0] + acc_ref[1]
        a0, a1, a2 = a[0], a[1], a[2]
        seg_s = a2[:, 0:96]
        s = 0.5 * (seg_s * _INV_SQRT_AVG_NEIGH + skip_ref[...])
        scal = s[:, 0:32]
        g1 = s[:, 32:64]
        g2 = s[:, 64:96]
        s_ref[...] = scal * jax.nn.sigmoid(scal)
        v_sh = a0[:, 0:96]
        t_sh = jnp.concatenate([a1, a0[:, 96:128]], axis=1)
        gate1 = g1 * jax.nn.sigmoid(g1)
        gate2 = g2 * jax.nn.sigmoid(g2)
        half_inv = 0.5 * _INV_SQRT_AVG_NEIGH
        v_ref[...] = (v_sh * half_inv) * jnp.concatenate([gate1] * 3, axis=1)
        t_ref[...] = (t_sh * half_inv) * jnp.concatenate([gate2] * 5, axis=1)

    return pl.pallas_call(
        body,
        grid=(n // cn,),
        in_specs=[
            pl.BlockSpec((2, 3, cn, 128), lambda i: (0, 0, i, 0)),
            pl.BlockSpec((cn, 96), lambda i: (i, 0)),
        ],
        out_specs=[
            pl.BlockSpec((cn, 32), lambda i: (i, 0)),
            pl.BlockSpec((cn, 96), lambda i: (i, 0)),
            pl.BlockSpec((cn, 160), lambda i: (i, 0)),
        ],
        out_shape=[
            jax.ShapeDtypeStruct((n, 32), jnp.float32),
            jax.ShapeDtypeStruct((n, 96), jnp.float32),
            jax.ShapeDtypeStruct((n, 160), jnp.float32),
        ],
    )(acc, skip)


def kernel(node_features, edge_features, radial_embedding, senders, receivers,
           node_species, W_up, W_r1, W_r2, W_down0, W_down1, W_down2, W_skip):
    n, d = node_features.shape
    e = senders.shape[0]
    del node_species

    h, skip = _node_matmuls(node_features, W_up, W_skip[0])
    hs = _sc_gather(h, senders.astype(jnp.int32))
    pay = _edge_payload(radial_embedding, edge_features, hs,
                        W_r1, W_r2, W_down0, W_down1, W_down2)
    n_pad = 10240
    zeros_init = jnp.zeros((n_pad // 16, 128), jnp.float32)
    acc = _sc_scatter(pay, receivers.astype(jnp.int32), zeros_init, n_pad)
    out_s, out_v_sh, out_t_sh = _gate(acc, skip, n)

    out_v = out_v_sh.reshape(n, 3, 32).transpose(0, 2, 1).reshape(n, 96)
    out_t = out_t_sh.reshape(n, 5, 32).transpose(0, 2, 1).reshape(n, 160)
    return jnp.concatenate([out_s, out_v, out_t], axis=1)


# speedup vs baseline: 52.7745x; 1.2080x over previous
import functools

import jax
import jax.numpy as jnp
from jax import lax
from jax.experimental import pallas as pl
from jax.experimental.pallas import tpu as pltpu
from jax.experimental.pallas import tpu_sc as plsc

_INV_SQRT_AVG_NEIGH = 1.0 / (16.0 ** 0.5)


def _node_matmuls(nf, w_up, w_skip0):
    n, d = nf.shape
    ks = w_skip0.shape[1]
    cn = 1000
    assert n % cn == 0

    def body(nf_ref, wu_ref, wsk_ref, h_ref, skip_ref):
        x = nf_ref[...]
        h_ref[...] = jnp.dot(x, wu_ref[...], preferred_element_type=jnp.float32)
        skip_ref[...] = jnp.dot(x, wsk_ref[...], preferred_element_type=jnp.float32)

    return pl.pallas_call(
        body,
        grid=(n // cn,),
        in_specs=[
            pl.BlockSpec((cn, d), lambda i: (i, 0)),
            pl.BlockSpec((d, d), lambda i: (0, 0)),
            pl.BlockSpec((d, ks), lambda i: (0, 0)),
        ],
        out_specs=[
            pl.BlockSpec((cn, d), lambda i: (i, 0)),
            pl.BlockSpec((cn, ks), lambda i: (i, 0)),
        ],
        out_shape=[
            jax.ShapeDtypeStruct((n, d), jnp.float32),
            jax.ShapeDtypeStruct((n, ks), jnp.float32),
        ],
    )(nf, w_up, w_skip0)


def _sc_gather(h, senders):
    n, d = h.shape
    e = senders.shape[0]
    nw = 32
    ch = 128
    n_ch = e // ch
    tmax = (n_ch + nw - 1) // nw
    assert e % ch == 0

    mesh = plsc.VectorSubcoreMesh(core_axis_name="c", subcore_axis_name="s")

    @functools.partial(
        pl.kernel,
        out_type=jax.ShapeDtypeStruct((e, d), jnp.float32),
        mesh=mesh,
        scratch_types=[
            pltpu.VMEM((3, 1, ch), jnp.int32),
            pltpu.VMEM((3, ch, d), jnp.float32),
            pltpu.SemaphoreType.DMA,
            pltpu.SemaphoreType.DMA,
            pltpu.SemaphoreType.DMA,
        ],
    )
    def k(h_hbm, s_hbm, out_hbm, idxb, rows, isem, gsem, osem):
        c = lax.axis_index("c")
        s = lax.axis_index("s")
        wid = s * 2 + c
        trips = (n_ch - wid + nw - 1) // nw

        def start_idx(t):
            @pl.when(t < trips)
            def _():
                eb = (wid + t * nw) * ch
                pltpu.async_copy(s_hbm.at[pl.ds(eb, ch)], idxb.at[t % 3, 0],
                                 isem)

        def wait_idx(t):
            @pl.when(t < trips)
            def _():
                pltpu.make_async_copy(s_hbm.at[pl.ds(0, ch)],
                                      idxb.at[t % 3, 0], isem).wait()

        def start_gather(t):
            @pl.when(t < trips)
            def _():
                pltpu.async_copy(h_hbm.at[idxb.at[t % 3, 0]], rows.at[t % 3],
                                 gsem)

        def wait_gather(t):
            @pl.when(t < trips)
            def _():
                pltpu.make_async_copy(h_hbm.at[idxb.at[t % 3, 0]],
                                      rows.at[t % 3], gsem).wait()

        def wait_store(t):
            @pl.when((t >= 0) & (t < trips))
            def _():
                pltpu.make_async_copy(rows.at[t % 3],
                                      out_hbm.at[pl.ds(0, ch)], osem).wait()

        start_idx(0)
        wait_idx(0)
        start_gather(0)
        start_idx(1)

        def body(t, carry):
            wait_gather(t)

            @pl.when(t < trips)
            def _():
                pltpu.async_copy(rows.at[t % 3],
                                 out_hbm.at[pl.ds((wid + t * nw) * ch, ch)],
                                 osem)

            wait_idx(t + 1)
            wait_store(t - 2)
            start_gather(t + 1)
            start_idx(t + 2)
            return carry

        lax.fori_loop(0, tmax, body, 0)
        wait_store(tmax - 2)
        wait_store(tmax - 1)

    return k(h, senders)


def _edge_payload(rad, ef, hs, w_r1, w_r2, w_d0, w_d1, w_d2):
    e, r = rad.shape
    d = hs.shape[1]
    ce = 2000
    assert e % ce == 0

    bf = jnp.bfloat16
    zero_d32 = jnp.zeros((d, 32), jnp.float32)
    zero_d96 = jnp.zeros((d, 96), jnp.float32)
    w_a = jnp.concatenate([w_d1, w_d1, w_d1, zero_d32], axis=1).astype(bf)
    w_b = jnp.concatenate([zero_d96, w_d2], axis=1).astype(bf)
    w_c = jnp.concatenate([w_d2, w_d2, w_d2, w_d2], axis=1).astype(bf)
    m = jnp.zeros((9, 384), jnp.float32)
    for i in range(3):
        m = m.at[1 + i, 32 * i:32 * (i + 1)].set(1.0)
    m = m.at[8, 96:128].set(1.0)
    for i in range(4):
        m = m.at[4 + i, 128 + 32 * i:160 + 32 * i].set(1.0)
    m = m.at[0, 256:384].set(1.0)

    def body(rad_ref, ef_ref, hs_ref, wr1_ref, wr2_ref, wa_ref, wb_ref,
             wc_ref, m_ref, pay_ref):
        radb = rad_ref[...]
        efb = ef_ref[...]
        hsb = hs_ref[...]
        hid = jnp.dot(radb, wr1_ref[...], preferred_element_type=jnp.float32)
        hid = hid * jax.nn.sigmoid(hid)
        w = jnp.dot(hid.astype(bf), wr2_ref[...].astype(bf),
                    preferred_element_type=jnp.float32)
        p0 = w[:, 0:d] * hsb
        p1 = (w[:, d:2 * d] * hsb).astype(bf)
        p2 = (w[:, 2 * d:3 * d] * hsb).astype(bf)
        shb = jnp.dot(efb, m_ref[...], preferred_element_type=jnp.float32)
        g0 = jnp.dot(p1, wa_ref[...], preferred_element_type=jnp.float32)
        g0 = g0 + jnp.dot(p2, wb_ref[...], preferred_element_type=jnp.float32)
        g1 = jnp.dot(p2, wc_ref[...], preferred_element_type=jnp.float32)
        pay_ref[0] = g0 * shb[:, 0:128]
        pay_ref[1] = g1 * shb[:, 128:256]
        pay_ref[2] = p0 * shb[:, 256:384]

    return pl.pallas_call(
        body,
        grid=(e // ce,),
        in_specs=[
            pl.BlockSpec((ce, r), lambda i: (i, 0)),
            pl.BlockSpec((ce, 9), lambda i: (i, 0)),
            pl.BlockSpec((ce, d), lambda i: (i, 0)),
            pl.BlockSpec((r, 8), lambda i: (0, 0)),
            pl.BlockSpec((8, 3 * d), lambda i: (0, 0)),
            pl.BlockSpec((d, 128), lambda i: (0, 0)),
            pl.BlockSpec((d, 128), lambda i: (0, 0)),
            pl.BlockSpec((d, 128), lambda i: (0, 0)),
            pl.BlockSpec((9, 384), lambda i: (0, 0)),
        ],
        out_specs=pl.BlockSpec((3, ce, 128), lambda i: (0, i, 0)),
        out_shape=jax.ShapeDtypeStruct((3, e, 128), jnp.float32),
    )(rad, ef, hs, w_r1, w_r2, w_a, w_b, w_c, m)


def _sc_scatter(pay, recv, zeros_init, n):
    ng, e, w = pay.shape
    ch = 80
    ns = 16
    epc = e // 2
    n_ch = epc // ch
    npt = n // ns
    assert e % (2 * ch) == 0 and n % ns == 0 and npt % 8 == 0

    mesh = plsc.VectorSubcoreMesh(core_axis_name="c", subcore_axis_name="s")

    @functools.partial(
        pl.kernel,
        out_type=jax.ShapeDtypeStruct((2, ng, n, w), jnp.float32),
        mesh=mesh,
        scratch_types=[
            pltpu.VMEM_SHARED((n, w), jnp.float32),
            pltpu.VMEM((3, 1, ch), jnp.int32),
            pltpu.VMEM((3, ch, w), jnp.float32),
            pltpu.SemaphoreType.DMA,
            pltpu.SemaphoreType.DMA,
        ],
    )
    def k(pay_hbm, recv_hbm, zero_hbm, out_hbm, acc, idxb, payb, lsem, ssem):
        c = lax.axis_index("c")
        s = lax.axis_index("s")
        ebase = c * epc
        trips = (n_ch - s + ns - 1) // ns
        tmax = (n_ch + ns - 1) // ns

        for g in range(ng):
            pltpu.sync_copy(zero_hbm, acc.at[pl.ds(s * npt, npt)])
            plsc.subcore_barrier()

            def start_loads(t, g=g):
                @pl.when(t < trips)
                def _():
                    b = t % 3
                    eb = ebase + (s + t * ns) * ch
                    pltpu.async_copy(recv_hbm.at[pl.ds(eb, ch)],
                                     idxb.at[b, 0], lsem)
                    pltpu.async_copy(pay_hbm.at[g, pl.ds(eb, ch)],
                                     payb.at[b], lsem)

            def wait_loads(t):
                @pl.when(t < trips)
                def _():
                    b = t % 3
                    pltpu.make_async_copy(recv_hbm.at[pl.ds(0, ch)],
                                          idxb.at[b, 0], lsem).wait()
                    pltpu.make_async_copy(pay_hbm.at[0, pl.ds(0, ch)],
                                          payb.at[b], lsem).wait()

            def wait_scat(t):
                @pl.when((t >= 0) & (t < trips))
                def _():
                    b = t % 3
                    pltpu.make_async_copy(payb.at[b], acc.at[idxb.at[b, 0]],
                                          ssem).wait()

            start_loads(0)

            def body(t, carry):
                wait_scat(t - 2)
                wait_loads(t)

                @pl.when(t < trips)
                def _():
                    b = t % 3
                    pltpu.async_copy(payb.at[b], acc.at[idxb.at[b, 0]],
                                     ssem, add=True)

                start_loads(t + 1)
                return carry

            lax.fori_loop(0, tmax, body, 0)
            wait_scat(tmax - 2)
            wait_scat(tmax - 1)
            plsc.subcore_barrier()
            pltpu.sync_copy(acc.at[pl.ds(s * npt, npt)],
                            out_hbm.at[c, g, pl.ds(s * npt, npt)])
            plsc.subcore_barrier()

    return k(pay, recv, zeros_init)


def _gate(acc, skip, w_d0, n):
    cn = 1000
    assert n % cn == 0

    def body(acc_ref, skip_ref, wd0_ref, s_ref, v_ref, t_ref):
        a = acc_ref[0] + acc_ref[1]
        a0, a1, a2 = a[0], a[1], a[2]
        seg_s = jnp.dot(a2, wd0_ref[...], preferred_element_type=jnp.float32)
        s = 0.5 * (seg_s * _INV_SQRT_AVG_NEIGH + skip_ref[...])
        scal = s[:, 0:32]
        g1 = s[:, 32:64]
        g2 = s[:, 64:96]
        s_ref[...] = scal * jax.nn.sigmoid(scal)
        v_sh = a0[:, 0:96]
        t_sh = jnp.concatenate([a1, a0[:, 96:128]], axis=1)
        gate1 = g1 * jax.nn.sigmoid(g1)
        gate2 = g2 * jax.nn.sigmoid(g2)
        half_inv = 0.5 * _INV_SQRT_AVG_NEIGH
        v_ref[...] = (v_sh * half_inv) * jnp.concatenate([gate1] * 3, axis=1)
        t_ref[...] = (t_sh * half_inv) * jnp.concatenate([gate2] * 5, axis=1)

    return pl.pallas_call(
        body,
        grid=(n // cn,),
        in_specs=[
            pl.BlockSpec((2, 3, cn, 128), lambda i: (0, 0, i, 0)),
            pl.BlockSpec((cn, 96), lambda i: (i, 0)),
            pl.BlockSpec((128, 96), lambda i: (0, 0)),
        ],
        out_specs=[
            pl.BlockSpec((cn, 32), lambda i: (i, 0)),
            pl.BlockSpec((cn, 96), lambda i: (i, 0)),
            pl.BlockSpec((cn, 160), lambda i: (i, 0)),
        ],
        out_shape=[
            jax.ShapeDtypeStruct((n, 32), jnp.float32),
            jax.ShapeDtypeStruct((n, 96), jnp.float32),
            jax.ShapeDtypeStruct((n, 160), jnp.float32),
        ],
    )(acc, skip, w_d0)


def kernel(node_features, edge_features, radial_embedding, senders, receivers,
           node_species, W_up, W_r1, W_r2, W_down0, W_down1, W_down2, W_skip):
    n, d = node_features.shape
    e = senders.shape[0]
    del node_species

    h, skip = _node_matmuls(node_features, W_up, W_skip[0])
    hs = _sc_gather(h, senders.astype(jnp.int32))
    pay = _edge_payload(radial_embedding, edge_features, hs,
                        W_r1, W_r2, W_down0, W_down1, W_down2)
    n_pad = 10240
    zeros_init = jnp.zeros((n_pad // 16, 128), jnp.float32)
    acc = _sc_scatter(pay, receivers.astype(jnp.int32), zeros_init, n_pad)
    out_s, out_v_sh, out_t_sh = _gate(acc, skip, W_down0, n)

    out_v = out_v_sh.reshape(n, 3, 32).transpose(0, 2, 1).reshape(n, 96)
    out_t = out_t_sh.reshape(n, 5, 32).transpose(0, 2, 1).reshape(n, 160)
    return jnp.concatenate([out_s, out_v, out_t], axis=1)


# speedup vs baseline: 55.6625x; 1.0547x over previous
import functools

import jax
import jax.numpy as jnp
from jax import lax
from jax.experimental import pallas as pl
from jax.experimental.pallas import tpu as pltpu
from jax.experimental.pallas import tpu_sc as plsc

_INV_SQRT_AVG_NEIGH = 1.0 / (16.0 ** 0.5)


def _node_matmuls(nf, w_up, w_skip0):
    n, d = nf.shape
    ks = w_skip0.shape[1]
    cn = 1000
    assert n % cn == 0

    def body(nf_ref, wu_ref, wsk_ref, h_ref, skip_ref):
        x = nf_ref[...]
        h_ref[...] = jnp.dot(x, wu_ref[...], preferred_element_type=jnp.float32)
        skip_ref[...] = jnp.dot(x, wsk_ref[...], preferred_element_type=jnp.float32)

    return pl.pallas_call(
        body,
        grid=(n // cn,),
        in_specs=[
            pl.BlockSpec((cn, d), lambda i: (i, 0)),
            pl.BlockSpec((d, d), lambda i: (0, 0)),
            pl.BlockSpec((d, ks), lambda i: (0, 0)),
        ],
        out_specs=[
            pl.BlockSpec((cn, d), lambda i: (i, 0)),
            pl.BlockSpec((cn, ks), lambda i: (i, 0)),
        ],
        out_shape=[
            jax.ShapeDtypeStruct((n, d), jnp.float32),
            jax.ShapeDtypeStruct((n, ks), jnp.float32),
        ],
    )(nf, w_up, w_skip0)


def _sc_gather(h, senders):
    n, d = h.shape
    e = senders.shape[0]
    nw = 32
    ch = 128
    n_ch = e // ch
    tmax = (n_ch + nw - 1) // nw
    assert e % ch == 0

    mesh = plsc.VectorSubcoreMesh(core_axis_name="c", subcore_axis_name="s")

    @functools.partial(
        pl.kernel,
        out_type=jax.ShapeDtypeStruct((e, d), jnp.float32),
        mesh=mesh,
        scratch_types=[
            pltpu.VMEM((2, 1, ch), jnp.int32),
            pltpu.VMEM((2, ch, d), jnp.float32),
            pltpu.SemaphoreType.DMA,
            pltpu.SemaphoreType.DMA,
        ],
    )
    def k(h_hbm, s_hbm, out_hbm, idxb, rows, isem, osem):
        c = lax.axis_index("c")
        s = lax.axis_index("s")
        wid = s * 2 + c
        trips = (n_ch - wid + nw - 1) // nw

        def start_idx(t):
            @pl.when(t < trips)
            def _():
                eb = (wid + t * nw) * ch
                pltpu.async_copy(s_hbm.at[pl.ds(eb, ch)], idxb.at[t % 2, 0],
                                 isem)

        def wait_idx(t):
            @pl.when(t < trips)
            def _():
                pltpu.make_async_copy(s_hbm.at[pl.ds(0, ch)],
                                      idxb.at[t % 2, 0], isem).wait()

        def wait_store(t):
            @pl.when((t >= 0) & (t < trips))
            def _():
                pltpu.make_async_copy(rows.at[t % 2],
                                      out_hbm.at[pl.ds(0, ch)], osem).wait()

        start_idx(0)

        def body(t, carry):
            b = t % 2
            wait_idx(t)
            start_idx(t + 1)
            wait_store(t - 2)

            @pl.when(t < trips)
            def _():
                pltpu.sync_copy(h_hbm.at[idxb.at[b, 0]], rows.at[b])
                pltpu.async_copy(rows.at[b],
                                 out_hbm.at[pl.ds((wid + t * nw) * ch, ch)],
                                 osem)

            return carry

        lax.fori_loop(0, tmax, body, 0)
        wait_store(tmax - 2)
        wait_store(tmax - 1)

    return k(h, senders)


def _edge_payload(rad, ef, hs, w_r1, w_r2, w_d0, w_d1, w_d2):
    e, r = rad.shape
    d = hs.shape[1]
    ce = 2000
    assert e % ce == 0

    bf = jnp.bfloat16
    zero_d32 = jnp.zeros((d, 32), jnp.float32)
    zero_d96 = jnp.zeros((d, 96), jnp.float32)
    w_a = jnp.concatenate([w_d1, w_d1, w_d1, zero_d32], axis=1).astype(bf)
    w_b = jnp.concatenate([zero_d96, w_d2], axis=1).astype(bf)
    w_c = jnp.concatenate([w_d2, w_d2, w_d2, w_d2], axis=1).astype(bf)
    w_dd = jnp.concatenate([w_d0, zero_d32], axis=1).astype(bf)
    m = jnp.zeros((9, 384), jnp.float32)
    for i in range(3):
        m = m.at[1 + i, 32 * i:32 * (i + 1)].set(1.0)
    m = m.at[8, 96:128].set(1.0)
    for i in range(4):
        m = m.at[4 + i, 128 + 32 * i:160 + 32 * i].set(1.0)
    m = m.at[0, 256:352].set(1.0)

    def body(rad_ref, ef_ref, hs_ref, wr1_ref, wr2_ref, wa_ref, wb_ref,
             wc_ref, wd_ref, m_ref, pay_ref):
        radb = rad_ref[...]
        efb = ef_ref[...]
        hsb = hs_ref[...]
        hid = jnp.dot(radb, wr1_ref[...], preferred_element_type=jnp.float32)
        hid = hid * jax.nn.sigmoid(hid)
        w = jnp.dot(hid.astype(bf), wr2_ref[...].astype(bf),
                    preferred_element_type=jnp.float32)
        p0 = (w[:, 0:d] * hsb).astype(bf)
        p1 = (w[:, d:2 * d] * hsb).astype(bf)
        p2 = (w[:, 2 * d:3 * d] * hsb).astype(bf)
        shb = jnp.dot(efb, m_ref[...], preferred_element_type=jnp.float32)
        g0 = jnp.dot(p1, wa_ref[...], preferred_element_type=jnp.float32)
        g0 = g0 + jnp.dot(p2, wb_ref[...], preferred_element_type=jnp.float32)
        g1 = jnp.dot(p2, wc_ref[...], preferred_element_type=jnp.float32)
        g2 = jnp.dot(p0, wd_ref[...], preferred_element_type=jnp.float32)
        pay_ref[0] = g0 * shb[:, 0:128]
        pay_ref[1] = g1 * shb[:, 128:256]
        pay_ref[2] = g2 * shb[:, 256:384]

    return pl.pallas_call(
        body,
        grid=(e // ce,),
        in_specs=[
            pl.BlockSpec((ce, r), lambda i: (i, 0)),
            pl.BlockSpec((ce, 9), lambda i: (i, 0)),
            pl.BlockSpec((ce, d), lambda i: (i, 0)),
            pl.BlockSpec((r, 8), lambda i: (0, 0)),
            pl.BlockSpec((8, 3 * d), lambda i: (0, 0)),
            pl.BlockSpec((d, 128), lambda i: (0, 0)),
            pl.BlockSpec((d, 128), lambda i: (0, 0)),
            pl.BlockSpec((d, 128), lambda i: (0, 0)),
            pl.BlockSpec((d, 128), lambda i: (0, 0)),
            pl.BlockSpec((9, 384), lambda i: (0, 0)),
        ],
        out_specs=pl.BlockSpec((3, ce, 128), lambda i: (0, i, 0)),
        out_shape=jax.ShapeDtypeStruct((3, e, 128), jnp.float32),
    )(rad, ef, hs, w_r1, w_r2, w_a, w_b, w_c, w_dd, m)


def _sc_scatter(pay, recv, zeros_init, n):
    ng, e, w = pay.shape
    ch = 128
    ns = 16
    epc = e // 2
    n_ch = epc // ch
    npt = n // ns
    assert e % (2 * ch) == 0 and n % ns == 0 and npt % 8 == 0

    mesh = plsc.VectorSubcoreMesh(core_axis_name="c", subcore_axis_name="s")

    @functools.partial(
        pl.kernel,
        out_type=jax.ShapeDtypeStruct((2, ng, n, w), jnp.float32),
        mesh=mesh,
        scratch_types=[
            pltpu.VMEM_SHARED((n, w), jnp.float32),
            pltpu.VMEM((2, 1, ch), jnp.int32),
            pltpu.VMEM((2, ch, w), jnp.float32),
            pltpu.SemaphoreType.DMA,
            pltpu.SemaphoreType.DMA,
        ],
    )
    def k(pay_hbm, recv_hbm, zero_hbm, out_hbm, acc, idxb, payb, lsem, ssem):
        c = lax.axis_index("c")
        s = lax.axis_index("s")
        ebase = c * epc
        trips = (n_ch - s + ns - 1) // ns
        tmax = (n_ch + ns - 1) // ns

        for g in range(ng):
            pltpu.sync_copy(zero_hbm, acc.at[pl.ds(s * npt, npt)])
            plsc.subcore_barrier()

            def start_loads(t, g=g):
                @pl.when(t < trips)
                def _():
                    b = t % 2
                    eb = ebase + (s + t * ns) * ch
                    pltpu.async_copy(recv_hbm.at[pl.ds(eb, ch)],
                                     idxb.at[b, 0], lsem)
                    pltpu.async_copy(pay_hbm.at[g, pl.ds(eb, ch)],
                                     payb.at[b], lsem)

            def wait_loads(t):
                @pl.when(t < trips)
                def _():
                    b = t % 2
                    pltpu.make_async_copy(recv_hbm.at[pl.ds(0, ch)],
                                          idxb.at[b, 0], lsem).wait()
                    pltpu.make_async_copy(pay_hbm.at[0, pl.ds(0, ch)],
                                          payb.at[b], lsem).wait()

            def wait_scat(t):
                @pl.when((t >= 0) & (t < trips))
                def _():
                    b = t % 2
                    pltpu.make_async_copy(payb.at[b], acc.at[idxb.at[b, 0]],
                                          ssem).wait()

            start_loads(0)

            def body(t, carry):
                wait_scat(t - 1)
                wait_loads(t)

                @pl.when(t < trips)
                def _():
                    b = t % 2
                    pltpu.async_copy(payb.at[b], acc.at[idxb.at[b, 0]],
                                     ssem, add=True)

                start_loads(t + 1)
                return carry

            lax.fori_loop(0, tmax, body, 0)
            wait_scat(tmax - 1)
            plsc.subcore_barrier()
            pltpu.sync_copy(acc.at[pl.ds(s * npt, npt)],
                            out_hbm.at[c, g, pl.ds(s * npt, npt)])
            plsc.subcore_barrier()

    return k(pay, recv, zeros_init)


def _gate(acc, acc2, skip, n):
    cn = 1000
    assert n % cn == 0

    def body(acc_ref, acc2_ref, skip_ref, s_ref, v_ref, t_ref):
        a = (acc_ref[0] + acc_ref[1]
             + acc2_ref[0] + acc2_ref[1])
        a0, a1, a2 = a[0], a[1], a[2]
        seg_s = a2[:, 0:96]
        s = 0.5 * (seg_s * _INV_SQRT_AVG_NEIGH + skip_ref[...])
        scal = s[:, 0:32]
        g1 = s[:, 32:64]
        g2 = s[:, 64:96]
        s_ref[...] = scal * jax.nn.sigmoid(scal)
        v_sh = a0[:, 0:96]
        t_sh = jnp.concatenate([a1, a0[:, 96:128]], axis=1)
        gate1 = g1 * jax.nn.sigmoid(g1)
        gate2 = g2 * jax.nn.sigmoid(g2)
        half_inv = 0.5 * _INV_SQRT_AVG_NEIGH
        v_ref[...] = (v_sh * half_inv) * jnp.concatenate([gate1] * 3, axis=1)
        t_ref[...] = (t_sh * half_inv) * jnp.concatenate([gate2] * 5, axis=1)

    return pl.pallas_call(
        body,
        grid=(n // cn,),
        in_specs=[
            pl.BlockSpec((2, 3, cn, 128), lambda i: (0, 0, i, 0)),
            pl.BlockSpec((2, 3, cn, 128), lambda i: (0, 0, i, 0)),
            pl.BlockSpec((cn, 96), lambda i: (i, 0)),
        ],
        out_specs=[
            pl.BlockSpec((cn, 32), lambda i: (i, 0)),
            pl.BlockSpec((cn, 96), lambda i: (i, 0)),
            pl.BlockSpec((cn, 160), lambda i: (i, 0)),
        ],
        out_shape=[
            jax.ShapeDtypeStruct((n, 32), jnp.float32),
            jax.ShapeDtypeStruct((n, 96), jnp.float32),
            jax.ShapeDtypeStruct((n, 160), jnp.float32),
        ],
    )(acc, acc2, skip)


def kernel(node_features, edge_features, radial_embedding, senders, receivers,
           node_species, W_up, W_r1, W_r2, W_down0, W_down1, W_down2, W_skip):
    n, d = node_features.shape
    e = senders.shape[0]
    del node_species

    h, skip = _node_matmuls(node_features, W_up, W_skip[0])
    e1 = 96000
    snd = senders.astype(jnp.int32)
    rcv = receivers.astype(jnp.int32)
    n_pad = 10240
    zeros_init = jnp.zeros((n_pad // 16, 128), jnp.float32)
    hs1 = _sc_gather(h, snd[:e1])
    pay1 = _edge_payload(radial_embedding[:e1], edge_features[:e1], hs1,
                         W_r1, W_r2, W_down0, W_down1, W_down2)
    hs2 = _sc_gather(h, snd[e1:])
    acc1 = _sc_scatter(pay1, rcv[:e1], zeros_init, n_pad)
    pay2 = _edge_payload(radial_embedding[e1:], edge_features[e1:], hs2,
                         W_r1, W_r2, W_down0, W_down1, W_down2)
    acc2 = _sc_scatter(pay2, rcv[e1:], zeros_init, n_pad)
    out_s, out_v_sh, out_t_sh = _gate(acc1, acc2, skip, n)

    out_v = out_v_sh.reshape(n, 3, 32).transpose(0, 2, 1).reshape(n, 96)
    out_t = out_t_sh.reshape(n, 5, 32).transpose(0, 2, 1).reshape(n, 160)
    return jnp.concatenate([out_s, out_v, out_t], axis=1)
